# Initial kernel scaffold; baseline (speedup 1.0000x reference)
import jax
import jax.numpy as jnp
from jax.experimental import pallas as pl


def kernel(token_ids, table, pe):
    raise NotImplementedError("write your pallas kernel here")


# speedup vs baseline: 3.2900x; 3.2900x over previous
import functools
import math

import jax
import jax.numpy as jnp
from jax import lax
from jax.experimental import pallas as pl
from jax.experimental.pallas import tpu as pltpu
from jax.experimental.pallas import tpu_sc as plsc

_VOCAB = 100000
_D = 64
_B = 4096
_S = 200
_SCALE = math.sqrt(float(_D))

_NC = 2
_NS = 16
_NW = _NC * _NS
_BPW = _B // _NW
_HALF = _S // 2


def _sc_body(tok_hbm, table_hbm, pe_hbm, out_hbm, idx_v, rows_v, pe_v,
             sem0, sem1):
    cid = lax.axis_index("c")
    sid = lax.axis_index("s")
    wid = sid * _NC + cid

    pltpu.sync_copy(pe_hbm, pe_v)
    pltpu.sync_copy(tok_hbm.at[pl.ds(wid * _BPW, _BPW)], idx_v)

    def body(g, carry):
        b = wid * _BPW + g
        cp0 = pltpu.async_copy(table_hbm.at[idx_v.at[g, 0]],
                               rows_v.at[pl.ds(0, _HALF)], sem0)
        cp1 = pltpu.async_copy(table_hbm.at[idx_v.at[g, 1]],
                               rows_v.at[pl.ds(_HALF, _HALF)], sem1)
        cp0.wait()
        cp1.wait()

        def fma(s2, c2):
            for ss in range(4):
                r = s2 * 4 + ss
                for j in range(_D // 16):
                    sl = pl.ds(j * 16, 16)
                    rows_v[r, sl] = rows_v[r, sl] * _SCALE + pe_v[r, sl]
            return c2

        lax.fori_loop(0, _S // 4, fma, 0, unroll=False)
        pltpu.sync_copy(rows_v, out_hbm.at[pl.ds(b * _S, _S)])
        return carry

    lax.fori_loop(0, _BPW, body, 0, unroll=False)


def kernel(token_ids, table, pe):
    tok = token_ids.astype(jnp.int32).reshape(_B, 2, _HALF)
    pe_s = pe[:_S].astype(jnp.float32)

    mesh = plsc.VectorSubcoreMesh(core_axis_name="c", subcore_axis_name="s")
    run = functools.partial(
        pl.kernel,
        mesh=mesh,
        compiler_params=pltpu.CompilerParams(use_tc_tiling_on_sc=False),
        out_type=jax.ShapeDtypeStruct((_B * _S, _D), jnp.float32),
        scratch_types=[
            pltpu.VMEM((_BPW, 2, _HALF), jnp.int32),
            pltpu.VMEM((_S, _D), jnp.float32),
            pltpu.VMEM((_S, _D), jnp.float32),
            pltpu.SemaphoreType.DMA,
            pltpu.SemaphoreType.DMA,
        ],
    )(_sc_body)
    out = run(tok, table, pe_s)
    return out.reshape(_B, _S, _D)
